# Initial kernel scaffold; baseline (speedup 1.0000x reference)
import jax
import jax.numpy as jnp
from jax.experimental import pallas as pl


def kernel(x, edge_index, edge_weight, W0, W1):
    raise NotImplementedError("write your pallas kernel here")


# speedup vs baseline: 6.6440x; 6.6440x over previous
import functools

import jax
import jax.numpy as jnp
from jax import lax
from jax.experimental import pallas as pl
from jax.experimental.pallas import tpu as pltpu
from jax.experimental.pallas import tpu_sc as plsc

N_NODES = 10000
N_EDGES = 320000
D_IN = 128
D_HID = 64
D_OUT = 32

NW = 32
E_PER_W = N_EDGES // NW
BK = 80
NB = E_PER_W // BK
N_PAD = 10240
ROWS_PER_TILE = N_PAD // 16


def _make_spmm(D):
    nfv = D // 16
    mesh = plsc.VectorSubcoreMesh(core_axis_name="c", subcore_axis_name="s")

    @functools.partial(
        pl.kernel,
        mesh=mesh,
        compiler_params=pltpu.CompilerParams(use_tc_tiling_on_sc=False),
        out_type=jax.ShapeDtypeStruct((2, N_PAD, D), jnp.float32),
        scratch_types=[
            pltpu.VMEM((NB, BK), jnp.int32),
            pltpu.VMEM((NB, BK), jnp.int32),
            pltpu.VMEM((NB, BK), jnp.float32),
            pltpu.VMEM((BK, D), jnp.float32),
            pltpu.VMEM_SHARED((N_PAD, D), jnp.float32),
            pltpu.SemaphoreType.DMA,
        ],
    )
    def spmm(src_hbm, dst_hbm, wgt_hbm, h_hbm, zeros_hbm, out_hbm,
             src_v, dst_v, wgt_v, rows_v, accum, sem):
        c = lax.axis_index("c")
        s = lax.axis_index("s")
        wid = c * 16 + s
        pltpu.sync_copy(src_hbm.at[wid], src_v)
        pltpu.sync_copy(dst_hbm.at[wid], dst_v)
        pltpu.sync_copy(wgt_hbm.at[wid], wgt_v)
        pltpu.sync_copy(zeros_hbm.at[pl.ds(s * ROWS_PER_TILE, ROWS_PER_TILE)],
                        accum.at[pl.ds(s * ROWS_PER_TILE, ROWS_PER_TILE)])
        plsc.subcore_barrier()

        def body(b, carry):
            pltpu.async_copy(h_hbm.at[src_v.at[b]], rows_v, sem).wait()
            for g in range(BK // 16):
                wv = wgt_v[b, pl.ds(g * 16, 16)]
                for e in range(16):
                    scale = lax.gather(
                        wv, jnp.full((16, 1), e, jnp.int32),
                        lax.GatherDimensionNumbers(
                            offset_dims=(), collapsed_slice_dims=(0,),
                            start_index_map=(0,)),
                        slice_sizes=(1,),
                        mode=lax.GatherScatterMode.PROMISE_IN_BOUNDS)
                    r = g * 16 + e
                    for f in range(nfv):
                        rows_v[r, pl.ds(f * 16, 16)] = (
                            rows_v[r, pl.ds(f * 16, 16)] * scale)
            pltpu.sync_copy(rows_v, accum.at[dst_v.at[b]], add=True)
            return carry

        lax.fori_loop(0, NB, body, 0)
        plsc.subcore_barrier()
        pltpu.sync_copy(accum.at[pl.ds(s * ROWS_PER_TILE, ROWS_PER_TILE)],
                        out_hbm.at[c, pl.ds(s * ROWS_PER_TILE, ROWS_PER_TILE)])

    return spmm


_spmm64 = _make_spmm(D_HID)
_spmm32 = _make_spmm(D_OUT)


def _mm_body(x_ref, w_ref, o_ref):
    o_ref[...] = jnp.dot(x_ref[...], w_ref[...],
                         preferred_element_type=jnp.float32)


def _mm(x, w, bm):
    m, k = x.shape
    _, n = w.shape
    return pl.pallas_call(
        _mm_body,
        grid=(m // bm,),
        in_specs=[pl.BlockSpec((bm, k), lambda i: (i, 0)),
                  pl.BlockSpec((k, n), lambda i: (0, 0))],
        out_specs=pl.BlockSpec((bm, n), lambda i: (i, 0)),
        out_shape=jax.ShapeDtypeStruct((m, n), jnp.float32),
    )(x, w)


def _l2_body(p0_ref, p1_ref, w_ref, o_ref):
    h = jnp.maximum(p0_ref[...] + p1_ref[...], 0.0)
    o_ref[...] = jnp.dot(h, w_ref[...], preferred_element_type=jnp.float32)


def _l2(p0, p1, w, bm, m):
    k = p0.shape[1]
    n = w.shape[1]
    return pl.pallas_call(
        _l2_body,
        grid=(m // bm,),
        in_specs=[pl.BlockSpec((bm, k), lambda i: (i, 0)),
                  pl.BlockSpec((bm, k), lambda i: (i, 0)),
                  pl.BlockSpec((k, n), lambda i: (0, 0))],
        out_specs=pl.BlockSpec((bm, n), lambda i: (i, 0)),
        out_shape=jax.ShapeDtypeStruct((m, n), jnp.float32),
    )(p0, p1, w)


def _dec_body(p0i_ref, p1i_ref, p0j_ref, p1j_ref, recon_ref, emb_ref):
    zi = p0i_ref[...] + p1i_ref[...]
    zj = p0j_ref[...] + p1j_ref[...]
    recon_ref[...] = lax.dot_general(zi, zj, (((1,), (1,)), ((), ())),
                                     preferred_element_type=jnp.float32)

    @pl.when(pl.program_id(1) == 0)
    def _():
        emb_ref[...] = zi


def _decoder(p0, p1, bm, bn, m):
    k = p0.shape[1]
    nbi = m // bm
    nbj = pl.cdiv(m, bn)
    return pl.pallas_call(
        _dec_body,
        grid=(nbi, nbj),
        in_specs=[pl.BlockSpec((bm, k), lambda i, j: (i, 0)),
                  pl.BlockSpec((bm, k), lambda i, j: (i, 0)),
                  pl.BlockSpec((bn, k), lambda i, j: (j, 0)),
                  pl.BlockSpec((bn, k), lambda i, j: (j, 0))],
        out_specs=[pl.BlockSpec((bm, bn), lambda i, j: (i, j)),
                   pl.BlockSpec((bm, k), lambda i, j: (i, 0))],
        out_shape=[jax.ShapeDtypeStruct((m, m), jnp.float32),
                   jax.ShapeDtypeStruct((m, k), jnp.float32)],
    )(p0, p1, p0, p1)


def kernel(x, edge_index, edge_weight, W0, W1):
    src = edge_index[0].astype(jnp.int32).reshape(NW, NB, BK)
    dst = edge_index[1].astype(jnp.int32).reshape(NW, NB, BK)
    wgt = edge_weight.reshape(NW, NB, BK)
    zeros64 = jnp.zeros((N_PAD, D_HID), jnp.float32)
    zeros32 = jnp.zeros((N_PAD, D_OUT), jnp.float32)

    h0 = _mm(x, W0, 1000)
    p = _spmm64(src, dst, wgt, h0, zeros64)
    h1 = _l2(p[0], p[1], W1, 1000, N_NODES)
    q = _spmm32(src, dst, wgt, h1, zeros32)
    recon, emb = _decoder(q[0], q[1], 2000, 2048, N_NODES)
    return (recon, emb)


# speedup vs baseline: 8.9158x; 1.3419x over previous
import functools

import jax
import jax.numpy as jnp
from jax import lax
from jax.experimental import pallas as pl
from jax.experimental.pallas import tpu as pltpu
from jax.experimental.pallas import tpu_sc as plsc

N_NODES = 10000
N_EDGES = 320000
D_IN = 128
D_HID = 64
D_OUT = 32

NW = 32
E_PER_W = N_EDGES // NW
BK = 80
NB = E_PER_W // BK
N_PAD = 10240
ROWS_PER_TILE = N_PAD // 16


def _make_spmm(D):
    nfv = D // 16
    mesh = plsc.VectorSubcoreMesh(core_axis_name="c", subcore_axis_name="s")

    @functools.partial(
        pl.kernel,
        mesh=mesh,
        compiler_params=pltpu.CompilerParams(use_tc_tiling_on_sc=False),
        out_type=jax.ShapeDtypeStruct((2, N_PAD, D), jnp.float32),
        scratch_types=[
            pltpu.VMEM((NB, BK), jnp.int32),
            pltpu.VMEM((NB, BK), jnp.int32),
            pltpu.VMEM((NB, BK), jnp.float32),
            pltpu.VMEM((BK, D), jnp.float32),
            pltpu.VMEM((BK, D), jnp.float32),
            pltpu.VMEM_SHARED((N_PAD, D), jnp.float32),
            pltpu.SemaphoreType.DMA,
            pltpu.SemaphoreType.DMA,
            pltpu.SemaphoreType.DMA,
            pltpu.SemaphoreType.DMA,
        ],
    )
    def spmm(src_hbm, dst_hbm, wgt_hbm, h_hbm, zeros_hbm, out_hbm,
             src_v, dst_v, wgt_v, rows0, rows1, accum,
             gsem0, gsem1, ssem0, ssem1):
        c = lax.axis_index("c")
        s = lax.axis_index("s")
        wid = c * 16 + s
        pltpu.sync_copy(src_hbm.at[wid], src_v)
        pltpu.sync_copy(dst_hbm.at[wid], dst_v)
        pltpu.sync_copy(wgt_hbm.at[wid], wgt_v)
        pltpu.sync_copy(zeros_hbm.at[pl.ds(s * ROWS_PER_TILE, ROWS_PER_TILE)],
                        accum.at[pl.ds(s * ROWS_PER_TILE, ROWS_PER_TILE)])
        plsc.subcore_barrier()

        def scale(buf, b):
            for g in range(BK // 16):
                wv = wgt_v[b, pl.ds(g * 16, 16)]
                for e in range(16):
                    sc = lax.gather(
                        wv, jnp.full((16, 1), e, jnp.int32),
                        lax.GatherDimensionNumbers(
                            offset_dims=(), collapsed_slice_dims=(0,),
                            start_index_map=(0,)),
                        slice_sizes=(1,),
                        mode=lax.GatherScatterMode.PROMISE_IN_BOUNDS)
                    r = g * 16 + e
                    for f in range(nfv):
                        buf[r, pl.ds(f * 16, 16)] = (
                            buf[r, pl.ds(f * 16, 16)] * sc)

        def issue_gather(b, buf, sem):
            pltpu.async_copy(h_hbm.at[src_v.at[b]], buf, sem)

        def wait_gather(b, buf, sem):
            pltpu.make_async_copy(h_hbm.at[src_v.at[b]], buf, sem).wait()

        issue_gather(0, rows0, gsem0)
        issue_gather(1, rows1, gsem1)

        @pl.loop(0, NB - 1, step=2)
        def _(b):
            wait_gather(b, rows0, gsem0)
            scale(rows0, b)
            pltpu.async_copy(rows0, accum.at[dst_v.at[b]], ssem0, add=True)
            wait_gather(b + 1, rows1, gsem1)
            scale(rows1, b + 1)
            pltpu.async_copy(rows1, accum.at[dst_v.at[b + 1]], ssem1, add=True)
            pltpu.make_async_copy(rows0, accum.at[dst_v.at[b]], ssem0).wait()
            issue_gather(b + 2, rows0, gsem0)

            @pl.when(b < NB - 3)
            def _():
                issue_gather(b + 3, rows1, gsem1)

            pltpu.make_async_copy(
                rows1, accum.at[dst_v.at[b + 1]], ssem1).wait()

        wait_gather(NB - 1, rows0, gsem0)
        scale(rows0, NB - 1)
        pltpu.sync_copy(rows0, accum.at[dst_v.at[NB - 1]], add=True)

        plsc.subcore_barrier()
        pltpu.sync_copy(accum.at[pl.ds(s * ROWS_PER_TILE, ROWS_PER_TILE)],
                        out_hbm.at[c, pl.ds(s * ROWS_PER_TILE, ROWS_PER_TILE)])

    return spmm


_spmm64 = _make_spmm(D_HID)
_spmm32 = _make_spmm(D_OUT)


def _mm_body(x_ref, w_ref, o_ref):
    o_ref[...] = jnp.dot(x_ref[...], w_ref[...],
                         preferred_element_type=jnp.float32)


def _mm(x, w, bm):
    m, k = x.shape
    _, n = w.shape
    return pl.pallas_call(
        _mm_body,
        grid=(m // bm,),
        in_specs=[pl.BlockSpec((bm, k), lambda i: (i, 0)),
                  pl.BlockSpec((k, n), lambda i: (0, 0))],
        out_specs=pl.BlockSpec((bm, n), lambda i: (i, 0)),
        out_shape=jax.ShapeDtypeStruct((m, n), jnp.float32),
    )(x, w)


def _l2_body(p0_ref, p1_ref, w_ref, o_ref):
    h = jnp.maximum(p0_ref[0] + p1_ref[0], 0.0)
    o_ref[...] = jnp.dot(h, w_ref[...], preferred_element_type=jnp.float32)


def _l2(p, w, bm, m):
    k = p.shape[2]
    n = w.shape[1]
    return pl.pallas_call(
        _l2_body,
        grid=(m // bm,),
        in_specs=[pl.BlockSpec((1, bm, k), lambda i: (0, i, 0)),
                  pl.BlockSpec((1, bm, k), lambda i: (1, i, 0)),
                  pl.BlockSpec((k, n), lambda i: (0, 0))],
        out_specs=pl.BlockSpec((bm, n), lambda i: (i, 0)),
        out_shape=jax.ShapeDtypeStruct((m, n), jnp.float32),
    )(p, p, w)


def _dec_body(p0i_ref, p1i_ref, p0j_ref, p1j_ref, recon_ref, emb_ref):
    zi = p0i_ref[0] + p1i_ref[0]
    zj = p0j_ref[0] + p1j_ref[0]
    recon_ref[...] = lax.dot_general(zi, zj, (((1,), (1,)), ((), ())),
                                     preferred_element_type=jnp.float32)

    @pl.when(pl.program_id(1) == 0)
    def _():
        emb_ref[...] = zi


def _decoder(q, bm, bn, m):
    k = q.shape[2]
    nbi = m // bm
    nbj = pl.cdiv(m, bn)
    return pl.pallas_call(
        _dec_body,
        grid=(nbi, nbj),
        in_specs=[pl.BlockSpec((1, bm, k), lambda i, j: (0, i, 0)),
                  pl.BlockSpec((1, bm, k), lambda i, j: (1, i, 0)),
                  pl.BlockSpec((1, bn, k), lambda i, j: (0, j, 0)),
                  pl.BlockSpec((1, bn, k), lambda i, j: (1, j, 0))],
        out_specs=[pl.BlockSpec((bm, bn), lambda i, j: (i, j)),
                   pl.BlockSpec((bm, k), lambda i, j: (i, 0))],
        out_shape=[jax.ShapeDtypeStruct((m, m), jnp.float32),
                   jax.ShapeDtypeStruct((m, k), jnp.float32)],
    )(q, q, q, q)


def kernel(x, edge_index, edge_weight, W0, W1):
    src = edge_index[0].astype(jnp.int32).reshape(NW, NB, BK)
    dst = edge_index[1].astype(jnp.int32).reshape(NW, NB, BK)
    wgt = edge_weight.reshape(NW, NB, BK)
    zeros64 = jnp.zeros((N_PAD, D_HID), jnp.float32)
    zeros32 = jnp.zeros((N_PAD, D_OUT), jnp.float32)

    h0 = _mm(x, W0, 1000)
    p = _spmm64(src, dst, wgt, h0, zeros64)
    h1 = _l2(p, W1, 1000, N_NODES)
    q = _spmm32(src, dst, wgt, h1, zeros32)
    recon, emb = _decoder(q, 2000, 2048, N_NODES)
    return (recon, emb)


# speedup vs baseline: 9.9906x; 1.1205x over previous
import functools

import jax
import jax.numpy as jnp
from jax import lax
from jax.experimental import pallas as pl
from jax.experimental.pallas import tpu as pltpu
from jax.experimental.pallas import tpu_sc as plsc

N_NODES = 10000
N_EDGES = 320000
D_IN = 128
D_HID = 64
D_OUT = 32

NW = 32
E_PER_W = N_EDGES // NW
BK = 80
NB = E_PER_W // BK
N_PAD = 10240
ROWS_PER_TILE = N_PAD // 16


def _make_spmm(D):
    nfv = D // 16
    mesh = plsc.VectorSubcoreMesh(core_axis_name="c", subcore_axis_name="s")

    @functools.partial(
        pl.kernel,
        mesh=mesh,
        compiler_params=pltpu.CompilerParams(use_tc_tiling_on_sc=False),
        out_type=jax.ShapeDtypeStruct((2, N_PAD, D), jnp.float32),
        scratch_types=[
            pltpu.VMEM((NB, BK), jnp.int32),
            pltpu.VMEM((NB, BK), jnp.int32),
            pltpu.VMEM((NB, BK), jnp.float32),
            pltpu.VMEM((BK, D), jnp.float32),
            pltpu.VMEM((BK, D), jnp.float32),
            pltpu.VMEM((BK, D), jnp.float32),
            pltpu.VMEM((BK, D), jnp.float32),
            pltpu.VMEM_SHARED((N_PAD, D), jnp.float32),
            pltpu.SemaphoreType.DMA,
            pltpu.SemaphoreType.DMA,
            pltpu.SemaphoreType.DMA,
            pltpu.SemaphoreType.DMA,
            pltpu.SemaphoreType.DMA,
            pltpu.SemaphoreType.DMA,
            pltpu.SemaphoreType.DMA,
            pltpu.SemaphoreType.DMA,
        ],
    )
    def spmm(src_hbm, dst_hbm, wgt_hbm, h_hbm, zeros_hbm, out_hbm,
             src_v, dst_v, wgt_v, rows0, rows1, rows2, rows3, accum,
             gsem0, gsem1, gsem2, gsem3, ssem0, ssem1, ssem2, ssem3):
        c = lax.axis_index("c")
        s = lax.axis_index("s")
        wid = c * 16 + s
        pltpu.sync_copy(src_hbm.at[wid], src_v)
        pltpu.sync_copy(dst_hbm.at[wid], dst_v)
        pltpu.sync_copy(wgt_hbm.at[wid], wgt_v)
        pltpu.sync_copy(zeros_hbm.at[pl.ds(s * ROWS_PER_TILE, ROWS_PER_TILE)],
                        accum.at[pl.ds(s * ROWS_PER_TILE, ROWS_PER_TILE)])
        plsc.subcore_barrier()

        def scale(buf, b):
            for g in range(BK // 16):
                wv = wgt_v[b, pl.ds(g * 16, 16)]
                for e in range(16):
                    sc = lax.gather(
                        wv, jnp.full((16, 1), e, jnp.int32),
                        lax.GatherDimensionNumbers(
                            offset_dims=(), collapsed_slice_dims=(0,),
                            start_index_map=(0,)),
                        slice_sizes=(1,),
                        mode=lax.GatherScatterMode.PROMISE_IN_BOUNDS)
                    r = g * 16 + e
                    for f in range(nfv):
                        buf[r, pl.ds(f * 16, 16)] = (
                            buf[r, pl.ds(f * 16, 16)] * sc)

        bufs = (rows0, rows1, rows2, rows3)
        gsems = (gsem0, gsem1, gsem2, gsem3)
        ssems = (ssem0, ssem1, ssem2, ssem3)
        R = 4

        def issue_gather(b, buf, sem):
            pltpu.async_copy(h_hbm.at[src_v.at[b]], buf, sem)

        def wait_gather(b, buf, sem):
            pltpu.make_async_copy(h_hbm.at[src_v.at[b]], buf, sem).wait()

        def issue_scatter(b, buf, sem):
            pltpu.async_copy(buf, accum.at[dst_v.at[b]], sem, add=True)

        def wait_scatter(b, buf, sem):
            pltpu.make_async_copy(buf, accum.at[dst_v.at[b]], sem).wait()

        for k in range(R):
            issue_gather(k, bufs[k], gsems[k])

        @pl.loop(0, NB - 1, step=R)
        def _(b):
            for k in range(R):
                t = b + k
                kp = (k - 1) % R
                wait_gather(t, bufs[k], gsems[k])
                scale(bufs[k], t)
                issue_scatter(t, bufs[k], ssems[k])

                @pl.when(t > 0)
                def _():
                    wait_scatter(t - 1, bufs[kp], ssems[kp])

                    @pl.when(t - 1 + R < NB)
                    def _():
                        issue_gather(t - 1 + R, bufs[kp], gsems[kp])

        kt = (NB - 1) % R
        wait_gather(NB - 1, bufs[kt], gsems[kt])
        scale(bufs[kt], NB - 1)
        wait_scatter(NB - 2, bufs[(NB - 2) % R], ssems[(NB - 2) % R])
        pltpu.sync_copy(bufs[kt], accum.at[dst_v.at[NB - 1]], add=True)

        plsc.subcore_barrier()
        pltpu.sync_copy(accum.at[pl.ds(s * ROWS_PER_TILE, ROWS_PER_TILE)],
                        out_hbm.at[c, pl.ds(s * ROWS_PER_TILE, ROWS_PER_TILE)])

    return spmm


_spmm64 = _make_spmm(D_HID)
_spmm32 = _make_spmm(D_OUT)


def _mm_body(x_ref, w_ref, o_ref):
    o_ref[...] = jnp.dot(x_ref[...], w_ref[...],
                         preferred_element_type=jnp.float32)


def _mm(x, w, bm):
    m, k = x.shape
    _, n = w.shape
    return pl.pallas_call(
        _mm_body,
        grid=(m // bm,),
        in_specs=[pl.BlockSpec((bm, k), lambda i: (i, 0)),
                  pl.BlockSpec((k, n), lambda i: (0, 0))],
        out_specs=pl.BlockSpec((bm, n), lambda i: (i, 0)),
        out_shape=jax.ShapeDtypeStruct((m, n), jnp.float32),
    )(x, w)


def _l2_body(p0_ref, p1_ref, w_ref, o_ref):
    h = jnp.maximum(p0_ref[0] + p1_ref[0], 0.0)
    o_ref[...] = jnp.dot(h, w_ref[...], preferred_element_type=jnp.float32)


def _l2(p, w, bm, m):
    k = p.shape[2]
    n = w.shape[1]
    return pl.pallas_call(
        _l2_body,
        grid=(m // bm,),
        in_specs=[pl.BlockSpec((1, bm, k), lambda i: (0, i, 0)),
                  pl.BlockSpec((1, bm, k), lambda i: (1, i, 0)),
                  pl.BlockSpec((k, n), lambda i: (0, 0))],
        out_specs=pl.BlockSpec((bm, n), lambda i: (i, 0)),
        out_shape=jax.ShapeDtypeStruct((m, n), jnp.float32),
    )(p, p, w)


def _dec_body(p0i_ref, p1i_ref, p0j_ref, p1j_ref, recon_ref, emb_ref):
    zi = p0i_ref[0] + p1i_ref[0]
    zj = p0j_ref[0] + p1j_ref[0]
    recon_ref[...] = lax.dot_general(zi, zj, (((1,), (1,)), ((), ())),
                                     preferred_element_type=jnp.float32)

    @pl.when(pl.program_id(1) == 0)
    def _():
        emb_ref[...] = zi


def _decoder(q, bm, bn, m):
    k = q.shape[2]
    nbi = m // bm
    nbj = pl.cdiv(m, bn)
    return pl.pallas_call(
        _dec_body,
        grid=(nbi, nbj),
        in_specs=[pl.BlockSpec((1, bm, k), lambda i, j: (0, i, 0)),
                  pl.BlockSpec((1, bm, k), lambda i, j: (1, i, 0)),
                  pl.BlockSpec((1, bn, k), lambda i, j: (0, j, 0)),
                  pl.BlockSpec((1, bn, k), lambda i, j: (1, j, 0))],
        out_specs=[pl.BlockSpec((bm, bn), lambda i, j: (i, j)),
                   pl.BlockSpec((bm, k), lambda i, j: (i, 0))],
        out_shape=[jax.ShapeDtypeStruct((m, m), jnp.float32),
                   jax.ShapeDtypeStruct((m, k), jnp.float32)],
    )(q, q, q, q)


def kernel(x, edge_index, edge_weight, W0, W1):
    src = edge_index[0].astype(jnp.int32).reshape(NW, NB, BK)
    dst = edge_index[1].astype(jnp.int32).reshape(NW, NB, BK)
    wgt = edge_weight.reshape(NW, NB, BK)
    zeros64 = jnp.zeros((N_PAD, D_HID), jnp.float32)
    zeros32 = jnp.zeros((N_PAD, D_OUT), jnp.float32)

    h0 = _mm(x, W0, 1000)
    p = _spmm64(src, dst, wgt, h0, zeros64)
    h1 = _l2(p, W1, 1000, N_NODES)
    q = _spmm32(src, dst, wgt, h1, zeros32)
    recon, emb = _decoder(q, 2000, 2048, N_NODES)
    return (recon, emb)


# speedup vs baseline: 10.2540x; 1.0264x over previous
import functools

import jax
import jax.numpy as jnp
from jax import lax
from jax.experimental import pallas as pl
from jax.experimental.pallas import tpu as pltpu
from jax.experimental.pallas import tpu_sc as plsc

N_NODES = 10000
N_EDGES = 320000
D_IN = 128
D_HID = 64
D_OUT = 32

NW = 32
E_PER_W = N_EDGES // NW
BK = 80
NB = E_PER_W // BK
N_PAD = 10240
ROWS_PER_TILE = N_PAD // 16


def _make_spmm(D):
    nfv = D // 16
    mesh = plsc.VectorSubcoreMesh(core_axis_name="c", subcore_axis_name="s")

    @functools.partial(
        pl.kernel,
        mesh=mesh,
        compiler_params=pltpu.CompilerParams(use_tc_tiling_on_sc=False),
        out_type=jax.ShapeDtypeStruct((2, N_PAD, D), jnp.float32),
        scratch_types=[
            pltpu.VMEM((NB, BK), jnp.int32),
            pltpu.VMEM((NB, BK), jnp.int32),
            pltpu.VMEM((NB, BK), jnp.float32),
            pltpu.VMEM((BK, D), jnp.float32),
            pltpu.VMEM((BK, D), jnp.float32),
            pltpu.VMEM((BK, D), jnp.float32),
            pltpu.VMEM((BK, D), jnp.float32),
            pltpu.VMEM((BK, D), jnp.float32),
            pltpu.VMEM_SHARED((N_PAD, D), jnp.float32),
            pltpu.SemaphoreType.DMA,
            pltpu.SemaphoreType.DMA,
            pltpu.SemaphoreType.DMA,
            pltpu.SemaphoreType.DMA,
            pltpu.SemaphoreType.DMA,
            pltpu.SemaphoreType.DMA,
            pltpu.SemaphoreType.DMA,
            pltpu.SemaphoreType.DMA,
            pltpu.SemaphoreType.DMA,
            pltpu.SemaphoreType.DMA,
        ],
    )
    def spmm(src_hbm, dst_hbm, wgt_hbm, h_hbm, zeros_hbm, out_hbm,
             src_v, dst_v, wgt_v, rows0, rows1, rows2, rows3, rows4, accum,
             gsem0, gsem1, gsem2, gsem3, gsem4,
             ssem0, ssem1, ssem2, ssem3, ssem4):
        c = lax.axis_index("c")
        s = lax.axis_index("s")
        wid = c * 16 + s
        pltpu.sync_copy(src_hbm.at[wid], src_v)
        pltpu.sync_copy(dst_hbm.at[wid], dst_v)
        pltpu.sync_copy(wgt_hbm.at[wid], wgt_v)
        pltpu.sync_copy(zeros_hbm.at[pl.ds(s * ROWS_PER_TILE, ROWS_PER_TILE)],
                        accum.at[pl.ds(s * ROWS_PER_TILE, ROWS_PER_TILE)])
        plsc.subcore_barrier()

        def scale(buf, b):
            for g in range(BK // 16):
                wv = wgt_v[b, pl.ds(g * 16, 16)]
                for e in range(16):
                    sc = lax.gather(
                        wv, jnp.full((16, 1), e, jnp.int32),
                        lax.GatherDimensionNumbers(
                            offset_dims=(), collapsed_slice_dims=(0,),
                            start_index_map=(0,)),
                        slice_sizes=(1,),
                        mode=lax.GatherScatterMode.PROMISE_IN_BOUNDS)
                    r = g * 16 + e
                    for f in range(nfv):
                        buf[r, pl.ds(f * 16, 16)] = (
                            buf[r, pl.ds(f * 16, 16)] * sc)

        bufs = (rows0, rows1, rows2, rows3, rows4)
        gsems = (gsem0, gsem1, gsem2, gsem3, gsem4)
        ssems = (ssem0, ssem1, ssem2, ssem3, ssem4)
        R = 5

        def issue_gather(b, buf, sem):
            pltpu.async_copy(h_hbm.at[src_v.at[b]], buf, sem)

        def wait_gather(b, buf, sem):
            pltpu.make_async_copy(h_hbm.at[src_v.at[b]], buf, sem).wait()

        def issue_scatter(b, buf, sem):
            pltpu.async_copy(buf, accum.at[dst_v.at[b]], sem, add=True)

        def wait_scatter(b, buf, sem):
            pltpu.make_async_copy(buf, accum.at[dst_v.at[b]], sem).wait()

        for k in range(R):
            issue_gather(k, bufs[k], gsems[k])

        @pl.loop(0, NB, step=R)
        def _(b):
            for k in range(R):
                t = b + k
                kp = (k - 1) % R
                wait_gather(t, bufs[k], gsems[k])
                scale(bufs[k], t)
                issue_scatter(t, bufs[k], ssems[k])

                @pl.when(t > 0)
                def _():
                    wait_scatter(t - 1, bufs[kp], ssems[kp])

                    @pl.when(t - 1 + R < NB)
                    def _():
                        issue_gather(t - 1 + R, bufs[kp], gsems[kp])

        wait_scatter(NB - 1, bufs[(NB - 1) % R], ssems[(NB - 1) % R])

        plsc.subcore_barrier()
        pltpu.sync_copy(accum.at[pl.ds(s * ROWS_PER_TILE, ROWS_PER_TILE)],
                        out_hbm.at[c, pl.ds(s * ROWS_PER_TILE, ROWS_PER_TILE)])

    return spmm


_spmm64 = _make_spmm(D_HID)
_spmm32 = _make_spmm(D_OUT)


def _mm_body(x_ref, w_ref, o_ref):
    o_ref[...] = jnp.dot(x_ref[...], w_ref[...],
                         preferred_element_type=jnp.float32)


def _mm(x, w, bm):
    m, k = x.shape
    _, n = w.shape
    return pl.pallas_call(
        _mm_body,
        grid=(m // bm,),
        in_specs=[pl.BlockSpec((bm, k), lambda i: (i, 0)),
                  pl.BlockSpec((k, n), lambda i: (0, 0))],
        out_specs=pl.BlockSpec((bm, n), lambda i: (i, 0)),
        out_shape=jax.ShapeDtypeStruct((m, n), jnp.float32),
    )(x, w)


def _l2_body(p0_ref, p1_ref, w_ref, o_ref):
    h = jnp.maximum(p0_ref[0] + p1_ref[0], 0.0)
    o_ref[...] = jnp.dot(h, w_ref[...], preferred_element_type=jnp.float32)


def _l2(p, w, bm, m):
    k = p.shape[2]
    n = w.shape[1]
    return pl.pallas_call(
        _l2_body,
        grid=(m // bm,),
        in_specs=[pl.BlockSpec((1, bm, k), lambda i: (0, i, 0)),
                  pl.BlockSpec((1, bm, k), lambda i: (1, i, 0)),
                  pl.BlockSpec((k, n), lambda i: (0, 0))],
        out_specs=pl.BlockSpec((bm, n), lambda i: (i, 0)),
        out_shape=jax.ShapeDtypeStruct((m, n), jnp.float32),
    )(p, p, w)


def _dec_body(p0i_ref, p1i_ref, p0j_ref, p1j_ref, recon_ref, emb_ref):
    zi = p0i_ref[0] + p1i_ref[0]
    zj = p0j_ref[0] + p1j_ref[0]
    recon_ref[...] = lax.dot_general(zi, zj, (((1,), (1,)), ((), ())),
                                     preferred_element_type=jnp.float32)

    @pl.when(pl.program_id(1) == 0)
    def _():
        emb_ref[...] = zi


def _decoder(q, bm, bn, m):
    k = q.shape[2]
    nbi = m // bm
    nbj = pl.cdiv(m, bn)
    return pl.pallas_call(
        _dec_body,
        grid=(nbi, nbj),
        in_specs=[pl.BlockSpec((1, bm, k), lambda i, j: (0, i, 0)),
                  pl.BlockSpec((1, bm, k), lambda i, j: (1, i, 0)),
                  pl.BlockSpec((1, bn, k), lambda i, j: (0, j, 0)),
                  pl.BlockSpec((1, bn, k), lambda i, j: (1, j, 0))],
        out_specs=[pl.BlockSpec((bm, bn), lambda i, j: (i, j)),
                   pl.BlockSpec((bm, k), lambda i, j: (i, 0))],
        out_shape=[jax.ShapeDtypeStruct((m, m), jnp.float32),
                   jax.ShapeDtypeStruct((m, k), jnp.float32)],
    )(q, q, q, q)


def kernel(x, edge_index, edge_weight, W0, W1):
    src = edge_index[0].astype(jnp.int32).reshape(NW, NB, BK)
    dst = edge_index[1].astype(jnp.int32).reshape(NW, NB, BK)
    wgt = edge_weight.reshape(NW, NB, BK)
    zeros64 = jnp.zeros((N_PAD, D_HID), jnp.float32)
    zeros32 = jnp.zeros((N_PAD, D_OUT), jnp.float32)

    h0 = _mm(x, W0, 1000)
    p = _spmm64(src, dst, wgt, h0, zeros64)
    h1 = _l2(p, W1, 1000, N_NODES)
    q = _spmm32(src, dst, wgt, h1, zeros32)
    recon, emb = _decoder(q, 2000, 2048, N_NODES)
    return (recon, emb)
